# Initial kernel scaffold; baseline (speedup 1.0000x reference)
import jax
import jax.numpy as jnp
from jax.experimental import pallas as pl


def kernel(x, w_gate, expert_bias, expert_weight):
    raise NotImplementedError("write your pallas kernel here")


# speedup vs baseline: 2.0751x; 2.0751x over previous
import functools

import jax
import jax.numpy as jnp
from jax.experimental import pallas as pl
from jax.experimental.pallas import tpu as pltpu

E = 8
K = 2


def _moe_body(x_ref, wg_ref, b_ref, w_ref, out_ref, gates_ref, *, n_tiles):
    n_i = pl.program_id(0)
    e_i = pl.program_id(1)

    @pl.when(jnp.logical_and(n_i == 0, e_i == 0))
    def _compute_gates():
        x = x_ref[...]
        logits = jax.lax.dot_general(
            x, wg_ref[...], (((1,), (0,)), ((), ())),
            preferred_element_type=jnp.float32)
        n, e = logits.shape
        iota = jax.lax.broadcasted_iota(jnp.int32, (n, e), 1)
        m1 = jnp.max(logits, axis=1, keepdims=True)
        i1 = jnp.min(jnp.where(logits == m1, iota, e), axis=1, keepdims=True)
        sel1 = iota == i1
        masked = jnp.where(sel1, -jnp.inf, logits)
        m2 = jnp.max(masked, axis=1, keepdims=True)
        i2 = jnp.min(jnp.where(masked == m2, iota, e), axis=1, keepdims=True)
        sel2 = iota == i2
        e2 = jnp.exp(m2 - m1)
        denom = 1.0 + e2
        gates_ref[...] = jnp.where(sel1, 1.0 / denom, 0.0) + jnp.where(
            sel2, e2 / denom, 0.0)

    x = x_ref[...]
    w = w_ref[0]
    b = b_ref[...]
    xw = jax.lax.dot_general(
        x, w, (((1,), (1,)), ((), ())), preferred_element_type=jnp.float32)
    row_iota = jax.lax.broadcasted_iota(jnp.int32, b.shape, 0)
    be = jnp.sum(jnp.where(row_iota == e_i, b, 0.0), axis=0, keepdims=True)
    bc = jax.lax.dot_general(
        be, w, (((1,), (1,)), ((), ())), preferred_element_type=jnp.float32)
    gates = gates_ref[...]
    col_iota = jax.lax.broadcasted_iota(jnp.int32, gates.shape, 1)
    g = jnp.sum(jnp.where(col_iota == e_i, gates, 0.0), axis=1, keepdims=True)
    contrib = g * (xw - bc)

    @pl.when(e_i == 0)
    def _init():
        out_ref[...] = contrib

    @pl.when(e_i != 0)
    def _acc():
        out_ref[...] = out_ref[...] + contrib


def kernel(x, w_gate, expert_bias, expert_weight):
    n_tok, d_in = x.shape
    e, d_out, _ = expert_weight.shape
    tn = 256
    n_tiles = d_out // tn
    body = functools.partial(_moe_body, n_tiles=n_tiles)
    return pl.pallas_call(
        body,
        grid=(n_tiles, e),
        in_specs=[
            pl.BlockSpec((n_tok, d_in), lambda n, ei: (0, 0)),
            pl.BlockSpec((d_in, e), lambda n, ei: (0, 0)),
            pl.BlockSpec((e, d_in), lambda n, ei: (0, 0)),
            pl.BlockSpec((1, tn, d_in), lambda n, ei: (ei, n, 0)),
        ],
        out_specs=pl.BlockSpec((n_tok, tn), lambda n, ei: (0, n)),
        out_shape=jax.ShapeDtypeStruct((n_tok, d_out), jnp.float32),
        scratch_shapes=[pltpu.VMEM((n_tok, e), jnp.float32)],
        compiler_params=pltpu.CompilerParams(
            dimension_semantics=("arbitrary", "arbitrary")),
    )(x, w_gate, expert_bias, expert_weight)


# speedup vs baseline: 2.0856x; 1.0051x over previous
import functools

import jax
import jax.numpy as jnp
from jax.experimental import pallas as pl
from jax.experimental.pallas import tpu as pltpu

E = 8
K = 2


def _moe_body(x_ref, wg_ref, b_ref, w_ref, out_ref, gates_ref, xb_ref, *,
              n_tiles):
    n_i = pl.program_id(0)
    e_i = pl.program_id(1)

    @pl.when(jnp.logical_and(n_i == 0, e_i == 0))
    def _compute_gates():
        x = x_ref[...]
        xb_ref[...] = x.astype(jnp.bfloat16)
        logits = jax.lax.dot_general(
            x, wg_ref[...], (((1,), (0,)), ((), ())),
            preferred_element_type=jnp.float32)
        n, e = logits.shape
        iota = jax.lax.broadcasted_iota(jnp.int32, (n, e), 1)
        m1 = jnp.max(logits, axis=1, keepdims=True)
        i1 = jnp.min(jnp.where(logits == m1, iota, e), axis=1, keepdims=True)
        sel1 = iota == i1
        masked = jnp.where(sel1, -jnp.inf, logits)
        m2 = jnp.max(masked, axis=1, keepdims=True)
        i2 = jnp.min(jnp.where(masked == m2, iota, e), axis=1, keepdims=True)
        sel2 = iota == i2
        e2 = jnp.exp(m2 - m1)
        denom = 1.0 + e2
        gates_ref[...] = jnp.where(sel1, 1.0 / denom, 0.0) + jnp.where(
            sel2, e2 / denom, 0.0)

    w = w_ref[0]
    b = b_ref[...]
    xw = jax.lax.dot_general(
        xb_ref[...], w.astype(jnp.bfloat16), (((1,), (1,)), ((), ())),
        preferred_element_type=jnp.float32)
    row_iota = jax.lax.broadcasted_iota(jnp.int32, b.shape, 0)
    be = jnp.sum(jnp.where(row_iota == e_i, b, 0.0), axis=0, keepdims=True)
    bc = jax.lax.dot_general(
        be, w, (((1,), (1,)), ((), ())), preferred_element_type=jnp.float32)
    gates = gates_ref[...]
    col_iota = jax.lax.broadcasted_iota(jnp.int32, gates.shape, 1)
    g = jnp.sum(jnp.where(col_iota == e_i, gates, 0.0), axis=1, keepdims=True)
    contrib = g * (xw - bc)

    @pl.when(e_i == 0)
    def _init():
        out_ref[...] = contrib

    @pl.when(e_i != 0)
    def _acc():
        out_ref[...] = out_ref[...] + contrib


def kernel(x, w_gate, expert_bias, expert_weight):
    n_tok, d_in = x.shape
    e, d_out, _ = expert_weight.shape
    tn = 256
    n_tiles = d_out // tn
    body = functools.partial(_moe_body, n_tiles=n_tiles)
    return pl.pallas_call(
        body,
        grid=(n_tiles, e),
        in_specs=[
            pl.BlockSpec((n_tok, d_in), lambda n, ei: (0, 0)),
            pl.BlockSpec((d_in, e), lambda n, ei: (0, 0)),
            pl.BlockSpec((e, d_in), lambda n, ei: (0, 0)),
            pl.BlockSpec((1, tn, d_in), lambda n, ei: (ei, n, 0)),
        ],
        out_specs=pl.BlockSpec((n_tok, tn), lambda n, ei: (0, n)),
        out_shape=jax.ShapeDtypeStruct((n_tok, d_out), jnp.float32),
        scratch_shapes=[pltpu.VMEM((n_tok, e), jnp.float32),
                        pltpu.VMEM((n_tok, d_in), jnp.bfloat16)],
        compiler_params=pltpu.CompilerParams(
            dimension_semantics=("arbitrary", "arbitrary")),
    )(x, w_gate, expert_bias, expert_weight)
